# Initial kernel scaffold; baseline (speedup 1.0000x reference)
import jax
import jax.numpy as jnp
from jax.experimental import pallas as pl


def kernel(input_word, context_word, input_ids, other_features, emb_in, emb_ctx, neg_idx, conv_w0, conv_b0, conv_w1, conv_b1, conv_w2, conv_b2, fc_w0, fc_b0, fc_w1, fc_b1, fc_w2, fc_b2, fc_w3, fc_b3, fc_w4, fc_b4, fc_w5, fc_b5):
    raise NotImplementedError("write your pallas kernel here")


# speedup vs baseline: 2.2115x; 2.2115x over previous
import functools

import jax
import jax.numpy as jnp
from jax import lax
from jax.experimental import pallas as pl
from jax.experimental.pallas import tpu as pltpu
from jax.experimental.pallas import tpu_sc as plsc

B = 4096
V = 100000
D = 64
L = 50
NEG = 10
NF = 20
FS = (3, 4, 5)

BB = 128
NB = B // BB
CH = 128
NW = 32


def _logsig(x):
    return jnp.minimum(x, 0.0) - jnp.log(1.0 + jnp.exp(-jnp.abs(x)))


def _tc_body(ein_ref, ectx_ref, eneg_ref, x_ref, oth_ref, wcat_ref,
             cb_ref, weff_ref, beff_ref, logits_ref, loss_ref):
    b = pl.program_id(0)

    ein = ein_ref[...]
    ectx = ectx_ref[...]
    prod = jnp.sum(ein * ectx, axis=1)
    pos = _logsig(prod)

    eneg = eneg_ref[...]
    nd = -jnp.sum(eneg * ein[None, :, :], axis=2)
    noise = jnp.sum(_logsig(nd), axis=0)

    part = jnp.sum(pos + noise)

    @pl.when(b == 0)
    def _():
        loss_ref[...] = jnp.zeros((1, 1), jnp.float32)

    loss_ref[...] = loss_ref[...] + part

    @pl.when(b == NB - 1)
    def _():
        loss_ref[...] = loss_ref[...] * (-1.0 / B)

    x = x_ref[...]
    x2 = x.reshape(L * BB, D)
    y = jnp.dot(x2, wcat_ref[...],
                preferred_element_type=jnp.float32)
    y3 = y.reshape(L, BB, sum(FS) * NF)

    cb = cb_ref[...]
    pools = []
    col = 0
    for i, fs in enumerate(FS):
        lout = L - fs + 1
        acc = y3[0:lout, :, col:col + NF]
        for k in range(1, fs):
            acc = acc + y3[k:k + lout, :, col + k * NF:col + (k + 1) * NF]
        col += fs * NF
        acc = jnp.maximum(acc + cb[i:i + 1, :][None, :, :], 0.0)
        pools.append(jnp.max(acc, axis=0))

    xf = jnp.concatenate(pools + [oth_ref[...]], axis=1)
    logits_ref[...] = (jnp.dot(xf, weff_ref[...],
                               preferred_element_type=jnp.float32)
                       + beff_ref[...])


def _tc_specs():
    ncol = sum(FS) * NF
    in_specs = [
        pl.BlockSpec((BB, D), lambda b: (b, 0)),
        pl.BlockSpec((BB, D), lambda b: (b, 0)),
        pl.BlockSpec((NEG, BB, D), lambda b: (0, b, 0)),
        pl.BlockSpec((L, BB, D), lambda b: (0, b, 0)),
        pl.BlockSpec((BB, 9), lambda b: (b, 0)),
        pl.BlockSpec((D, ncol), lambda b: (0, 0)),
        pl.BlockSpec((3, NF), lambda b: (0, 0)),
        pl.BlockSpec((69, 2), lambda b: (0, 0)),
        pl.BlockSpec((1, 2), lambda b: (0, 0)),
    ]
    out_specs = [
        pl.BlockSpec((BB, 2), lambda b: (b, 0)),
        pl.BlockSpec((1, 1), lambda b: (0, 0)),
    ]
    out_shape = [
        jax.ShapeDtypeStruct((B, 2), jnp.float32),
        jax.ShapeDtypeStruct((1, 1), jnp.float32),
    ]
    return (NB,), in_specs, out_specs, out_shape


def _dense(ein_g, ectx_g, eneg3, x3, other, wcat, cb, weff, beff):
    grid, in_specs, out_specs, out_shape = _tc_specs()
    return pl.pallas_call(
        _tc_body, grid=grid, in_specs=in_specs, out_specs=out_specs,
        out_shape=out_shape,
    )(ein_g, ectx_g, eneg3, x3, other, wcat, cb, weff, beff)


def _sc_gather_build():
    xpw = (L * B) // NW
    npw = (NEG * B) // NW
    bpw = B // NW
    nx = xpw // CH
    nn = npw // CH

    mesh = plsc.VectorSubcoreMesh(core_axis_name="c", subcore_axis_name="s")

    @functools.partial(
        pl.kernel,
        mesh=mesh,
        compiler_params=pltpu.CompilerParams(use_tc_tiling_on_sc=False),
        out_type=[
            jax.ShapeDtypeStruct((B, D), jnp.float32),
            jax.ShapeDtypeStruct((L * B, D), jnp.float32),
            jax.ShapeDtypeStruct((B, D), jnp.float32),
            jax.ShapeDtypeStruct((NEG * B, D), jnp.float32),
        ],
        scratch_types=[
            pltpu.VMEM((xpw,), jnp.int32),
            pltpu.VMEM((npw,), jnp.int32),
            pltpu.VMEM((bpw,), jnp.int32),
            pltpu.VMEM((bpw,), jnp.int32),
            pltpu.VMEM((CH, D), jnp.float32),
            pltpu.VMEM((CH, D), jnp.float32),
            pltpu.VMEM((CH, D), jnp.float32),
            pltpu.VMEM((CH, D), jnp.float32),
            pltpu.SemaphoreType.DMA,
            pltpu.SemaphoreType.DMA,
            pltpu.SemaphoreType.DMA,
            pltpu.SemaphoreType.DMA,
        ],
    )
    def sc(emb_in_h, emb_ctx_h, iw_h, idsT_h, cw_h, negT_h,
           ein_o, x_o, ectx_o, eneg_o,
           ix_v, in_v, iw_v, ic_v, buf0, buf1, bufA, bufC,
           sem0, sem1, semA, semC):
        w = lax.axis_index("s") * 2 + lax.axis_index("c")

        pltpu.sync_copy(idsT_h.at[pl.ds(w * xpw, xpw)], ix_v)
        pltpu.sync_copy(negT_h.at[pl.ds(w * npw, npw)], in_v)
        pltpu.sync_copy(iw_h.at[pl.ds(w * bpw, bpw)], iw_v)
        pltpu.sync_copy(cw_h.at[pl.ds(w * bpw, bpw)], ic_v)

        cpA = pltpu.make_async_copy(emb_in_h.at[iw_v], bufA, semA)
        cpC = pltpu.make_async_copy(emb_ctx_h.at[ic_v], bufC, semC)
        cpA.start()
        cpC.start()

        bufs = (buf0, buf1)
        sems = (sem0, sem1)

        def run(table_h, idx_v, out_h, base, nch):
            def mk(slot, t):
                return pltpu.make_async_copy(
                    table_h.at[idx_v.at[pl.ds(t * CH, CH)]],
                    bufs[slot], sems[slot])

            mk(0, 0).start()
            mk(1, 1).start()

            def body(i, carry):
                t0 = i * 2
                for s in (0, 1):
                    t = t0 + s
                    mk(s, t).wait()
                    pltpu.sync_copy(bufs[s],
                                    out_h.at[pl.ds(base + t * CH, CH)])

                    @pl.when(t + 2 < nch)
                    def _():
                        mk(s, t + 2).start()
                return carry

            lax.fori_loop(0, nch // 2, body, 0)

        run(emb_in_h, ix_v, x_o, w * xpw, nx)

        cpA.wait()
        pltpu.sync_copy(bufA, ein_o.at[pl.ds(w * bpw, bpw)])
        cpC.wait()
        pltpu.sync_copy(bufC, ectx_o.at[pl.ds(w * bpw, bpw)])

        run(emb_ctx_h, in_v, eneg_o, w * npw, nn)

    return sc


def kernel(input_word, context_word, input_ids, other_features, emb_in,
           emb_ctx, neg_idx, conv_w0, conv_b0, conv_w1, conv_b1, conv_w2,
           conv_b2, fc_w0, fc_b0, fc_w1, fc_b1, fc_w2, fc_b2, fc_w3,
           fc_b3, fc_w4, fc_b4, fc_w5, fc_b5):
    iw = input_word.astype(jnp.int32)
    cw = context_word.astype(jnp.int32)
    idsT = input_ids.T.reshape(-1).astype(jnp.int32)
    negT = neg_idx.T.reshape(-1).astype(jnp.int32)

    sc = _sc_gather_build()
    ein_g, x_g, ectx_g, eneg_g = sc(emb_in, emb_ctx, iw, idsT, cw, negT)
    x3 = x_g.reshape(L, B, D)
    eneg3 = eneg_g.reshape(NEG, B, D)

    cols = ([conv_w0[:, :, k].T for k in range(FS[0])]
            + [conv_w1[:, :, k].T for k in range(FS[1])]
            + [conv_w2[:, :, k].T for k in range(FS[2])])
    wcat = jnp.concatenate(cols, axis=1)
    cb = jnp.stack([conv_b0, conv_b1, conv_b2], axis=0)

    weff = fc_w0
    beff = fc_b0
    for wi, bi in ((fc_w1, fc_b1), (fc_w2, fc_b2), (fc_w3, fc_b3),
                   (fc_w4, fc_b4), (fc_w5, fc_b5)):
        weff = weff @ wi
        beff = beff @ wi + bi
    beff = beff.reshape(1, 2)

    logits, loss_arr = _dense(ein_g, ectx_g, eneg3, x3, other_features,
                              wcat, cb, weff, beff)
    return (loss_arr[0, 0], logits)


# speedup vs baseline: 2.5858x; 1.1692x over previous
import functools

import jax
import jax.numpy as jnp
from jax import lax
from jax.experimental import pallas as pl
from jax.experimental.pallas import tpu as pltpu
from jax.experimental.pallas import tpu_sc as plsc

B = 4096
V = 100000
D = 64
L = 50
NEG = 10
NF = 20
FS = (3, 4, 5)
NCOL = sum(FS) * NF

BB = 128
HB = BB // 2
NB = B // BB
CH = 128
NW = 32


def _logsig(x):
    return jnp.minimum(x, 0.0) - jnp.log(1.0 + jnp.exp(-jnp.abs(x)))


def _tc_body(ein_ref, ectx_ref, eneg_ref, x_ref, othe_ref, otho_ref,
             w2_ref, cb_ref, weff_ref, beff_ref, h_ref,
             le_ref, lo_ref, loss_ref):
    b = pl.program_id(0)
    hmat = h_ref[...]

    ein = ein_ref[...]
    ectx = ectx_ref[...]
    prod = jnp.dot(ein * ectx, hmat,
                   preferred_element_type=jnp.float32)
    pos = _logsig(prod)

    eneg = eneg_ref[...]
    m = (eneg * ein[None, :, :]).reshape(NEG * HB, 128)
    nd = -jnp.dot(m, hmat, preferred_element_type=jnp.float32)
    noise = jnp.sum(_logsig(nd).reshape(NEG, HB, 2), axis=0)

    part = jnp.sum(pos + noise)

    @pl.when(b == 0)
    def _():
        loss_ref[...] = jnp.zeros((1, 1), jnp.float32)

    loss_ref[...] = loss_ref[...] + part

    @pl.when(b == NB - 1)
    def _():
        loss_ref[...] = loss_ref[...] * (-1.0 / B)

    x = x_ref[...]
    x2 = x.reshape(L * HB, 128)
    y = jnp.dot(x2, w2_ref[...],
                preferred_element_type=jnp.float32)
    y3 = y.reshape(L, HB, 2 * NCOL)

    cb = cb_ref[...]
    halves = ([], [])
    for h in (0, 1):
        col = h * NCOL
        for i, fs in enumerate(FS):
            lout = L - fs + 1
            acc = y3[0:lout, :, col:col + NF]
            for k in range(1, fs):
                acc = acc + y3[k:k + lout, :,
                               col + k * NF:col + (k + 1) * NF]
            col += fs * NF
            acc = jnp.maximum(acc + cb[i:i + 1, :][None, :, :], 0.0)
            halves[h].append(jnp.max(acc, axis=0))

    weff = weff_ref[...]
    beff = beff_ref[...]
    xfe = jnp.concatenate(halves[0] + [othe_ref[...]], axis=1)
    xfo = jnp.concatenate(halves[1] + [otho_ref[...]], axis=1)
    le_ref[...] = jnp.dot(xfe, weff,
                          preferred_element_type=jnp.float32) + beff
    lo_ref[...] = jnp.dot(xfo, weff,
                          preferred_element_type=jnp.float32) + beff


def _tc_specs():
    in_specs = [
        pl.BlockSpec((HB, 128), lambda b: (b, 0)),
        pl.BlockSpec((HB, 128), lambda b: (b, 0)),
        pl.BlockSpec((NEG, HB, 128), lambda b: (0, b, 0)),
        pl.BlockSpec((L, HB, 128), lambda b: (0, b, 0)),
        pl.BlockSpec((HB, 9), lambda b: (b, 0)),
        pl.BlockSpec((HB, 9), lambda b: (b, 0)),
        pl.BlockSpec((128, 2 * NCOL), lambda b: (0, 0)),
        pl.BlockSpec((3, NF), lambda b: (0, 0)),
        pl.BlockSpec((69, 2), lambda b: (0, 0)),
        pl.BlockSpec((1, 2), lambda b: (0, 0)),
        pl.BlockSpec((128, 2), lambda b: (0, 0)),
    ]
    out_specs = [
        pl.BlockSpec((HB, 2), lambda b: (b, 0)),
        pl.BlockSpec((HB, 2), lambda b: (b, 0)),
        pl.BlockSpec((1, 1), lambda b: (0, 0)),
    ]
    out_shape = [
        jax.ShapeDtypeStruct((B // 2, 2), jnp.float32),
        jax.ShapeDtypeStruct((B // 2, 2), jnp.float32),
        jax.ShapeDtypeStruct((1, 1), jnp.float32),
    ]
    return (NB,), in_specs, out_specs, out_shape


def _dense(ein_pk, ectx_pk, eneg_pk, x_pk, othe, otho, w2, cb, weff,
           beff, hmat):
    grid, in_specs, out_specs, out_shape = _tc_specs()
    return pl.pallas_call(
        _tc_body, grid=grid, in_specs=in_specs, out_specs=out_specs,
        out_shape=out_shape,
    )(ein_pk, ectx_pk, eneg_pk, x_pk, othe, otho, w2, cb, weff, beff,
      hmat)


def _sc_gather_build():
    xpw = (L * B) // NW
    npw = (NEG * B) // NW
    bpw = B // NW
    nx = xpw // CH
    nn = npw // CH
    PCH = CH // 2

    mesh = plsc.VectorSubcoreMesh(core_axis_name="c", subcore_axis_name="s")

    @functools.partial(
        pl.kernel,
        mesh=mesh,
        compiler_params=pltpu.CompilerParams(use_tc_tiling_on_sc=False),
        out_type=[
            jax.ShapeDtypeStruct((B, D), jnp.float32),
            jax.ShapeDtypeStruct((L * B, D), jnp.float32),
            jax.ShapeDtypeStruct((B, D), jnp.float32),
            jax.ShapeDtypeStruct((NEG * B, D), jnp.float32),
        ],
        scratch_types=[
            pltpu.VMEM((xpw,), jnp.int32),
            pltpu.VMEM((npw,), jnp.int32),
            pltpu.VMEM((bpw,), jnp.int32),
            pltpu.VMEM((bpw,), jnp.int32),
            pltpu.VMEM((CH, D), jnp.float32),
            pltpu.VMEM((CH, D), jnp.float32),
            pltpu.VMEM((CH, D), jnp.float32),
            pltpu.VMEM((CH, D), jnp.float32),
            pltpu.SemaphoreType.DMA,
            pltpu.SemaphoreType.DMA,
            pltpu.SemaphoreType.DMA,
            pltpu.SemaphoreType.DMA,
        ],
    )
    def sc(emb_in_h, emb_ctx_h, iw_h, idsT_h, cw_h, negT_h,
           ein_o, x_o, ectx_o, eneg_o,
           ix_v, in_v, iw_v, ic_v, buf0, buf1, bufA, bufC,
           sem0, sem1, semA, semC):
        w = lax.axis_index("s") * 2 + lax.axis_index("c")

        pltpu.sync_copy(idsT_h.at[pl.ds(w * xpw, xpw)], ix_v)
        pltpu.sync_copy(negT_h.at[pl.ds(w * npw, npw)], in_v)
        pltpu.sync_copy(iw_h.at[pl.ds(w * bpw, bpw)], iw_v)
        pltpu.sync_copy(cw_h.at[pl.ds(w * bpw, bpw)], ic_v)

        cpA = pltpu.make_async_copy(emb_in_h.at[iw_v], bufA, semA)
        cpC = pltpu.make_async_copy(emb_ctx_h.at[ic_v], bufC, semC)
        cpA.start()
        cpC.start()

        bufs = (buf0, buf1)
        sems = (sem0, sem1)

        def run(table_h, idx_v, out_h, base, nch):
            def mk(slot, t):
                return pltpu.make_async_copy(
                    table_h.at[idx_v.at[pl.ds(t * CH, CH)]],
                    bufs[slot], sems[slot])

            mk(0, 0).start()
            mk(1, 1).start()

            def body(i, carry):
                t0 = i * 2
                for s in (0, 1):
                    t = t0 + s
                    mk(s, t).wait()
                    pltpu.sync_copy(bufs[s],
                                    out_h.at[pl.ds(base + t * CH, CH)])

                    @pl.when(t + 2 < nch)
                    def _():
                        mk(s, t + 2).start()
                return carry

            lax.fori_loop(0, nch // 2, body, 0)

        run(emb_in_h, ix_v, x_o, w * xpw, nx)

        cpA.wait()
        pltpu.sync_copy(bufA, ein_o.at[pl.ds(w * bpw, bpw)])
        cpC.wait()
        pltpu.sync_copy(bufC, ectx_o.at[pl.ds(w * bpw, bpw)])

        run(emb_ctx_h, in_v, eneg_o, w * npw, nn)

    return sc


def kernel(input_word, context_word, input_ids, other_features, emb_in,
           emb_ctx, neg_idx, conv_w0, conv_b0, conv_w1, conv_b1, conv_w2,
           conv_b2, fc_w0, fc_b0, fc_w1, fc_b1, fc_w2, fc_b2, fc_w3,
           fc_b3, fc_w4, fc_b4, fc_w5, fc_b5):
    iw = input_word.astype(jnp.int32)
    cw = context_word.astype(jnp.int32)
    idsT = input_ids.T.reshape(-1).astype(jnp.int32)
    negT = neg_idx.T.reshape(-1).astype(jnp.int32)

    sc = _sc_gather_build()
    ein_g, x_g, ectx_g, eneg_g = sc(emb_in, emb_ctx, iw, idsT, cw, negT)
    ein_pk = ein_g.reshape(B // 2, 128)
    ectx_pk = ectx_g.reshape(B // 2, 128)
    x3 = x_g.reshape(L, B // 2, 128)
    eneg3 = eneg_g.reshape(NEG, B // 2, 128)

    cols = ([conv_w0[:, :, k].T for k in range(FS[0])]
            + [conv_w1[:, :, k].T for k in range(FS[1])]
            + [conv_w2[:, :, k].T for k in range(FS[2])])
    wcat = jnp.concatenate(cols, axis=1)
    z = jnp.zeros((D, NCOL), jnp.float32)
    w2 = jnp.concatenate([jnp.concatenate([wcat, z], axis=1),
                          jnp.concatenate([z, wcat], axis=1)], axis=0)
    cb = jnp.stack([conv_b0, conv_b1, conv_b2], axis=0)

    weff = fc_w0
    beff = fc_b0
    for wi, bi in ((fc_w1, fc_b1), (fc_w2, fc_b2), (fc_w3, fc_b3),
                   (fc_w4, fc_b4), (fc_w5, fc_b5)):
        weff = weff @ wi
        beff = beff @ wi + bi
    beff = beff.reshape(1, 2)

    hmat = (jnp.arange(128)[:, None] // 64
            == jnp.arange(2)[None, :]).astype(jnp.float32)

    othe = other_features[0::2]
    otho = other_features[1::2]

    le, lo, loss_arr = _dense(ein_pk, ectx_pk, eneg3, x3, othe, otho,
                              w2, cb, weff, beff, hmat)
    logits = jnp.stack([le, lo], axis=1).reshape(B, 2)
    return (loss_arr[0, 0], logits)


# speedup vs baseline: 4.4599x; 1.7248x over previous
import functools

import jax
import jax.numpy as jnp
from jax import lax
from jax.experimental import pallas as pl
from jax.experimental.pallas import tpu as pltpu
from jax.experimental.pallas import tpu_sc as plsc

B = 4096
V = 100000
D = 64
L = 50
NEG = 10
NF = 20
FS = (3, 4, 5)
NCOL = sum(FS) * NF

BB = 128
HB = BB // 2
NB = B // BB
CH = 128
NW = 32


def _logsig(x):
    return jnp.minimum(x, 0.0) - jnp.log(1.0 + jnp.exp(-jnp.abs(x)))


def _tc_body(ein_ref, ectx_ref, eneg_ref, x_ref, othe_ref, otho_ref,
             w5_ref, b120_ref, msk_ref, weff_ref, beff_ref, h_ref,
             le_ref, lo_ref, loss_ref):
    b = pl.program_id(0)
    hmat = h_ref[...]

    ein = ein_ref[...]
    ectx = ectx_ref[...]
    prod = jnp.dot(ein * ectx, hmat,
                   preferred_element_type=jnp.float32)
    pos = _logsig(prod)

    eneg = eneg_ref[...]
    m = (eneg * ein[None, :, :]).reshape(NEG * HB, 128)
    nd = -jnp.dot(m, hmat, preferred_element_type=jnp.float32)
    noise = jnp.sum(_logsig(nd).reshape(NEG, HB, 2), axis=0)

    part = jnp.sum(pos + noise)

    @pl.when(b == 0)
    def _():
        loss_ref[...] = jnp.zeros((1, 1), jnp.float32)

    loss_ref[...] = loss_ref[...] + part

    @pl.when(b == NB - 1)
    def _():
        loss_ref[...] = loss_ref[...] * (-1.0 / B)

    x = x_ref[...]
    x2 = x.reshape(L * HB, 128)
    LO = L - FS[0] + 1
    y5 = None
    for k in range(FS[-1]):
        yk = jnp.dot(x2, w5_ref[k],
                     preferred_element_type=jnp.float32)
        yk = yk.reshape(L, HB, 2 * 3 * NF)
        if k + LO <= L:
            sh = yk[k:k + LO]
        else:
            sh = jnp.concatenate(
                [yk[k:L],
                 jnp.zeros((k + LO - L, HB, 2 * 3 * NF), jnp.float32)],
                axis=0)
        y5 = sh if y5 is None else y5 + sh

    y5 = jnp.maximum(y5 + b120_ref[...][None, :, :], 0.0)
    y5 = y5 * msk_ref[...][:, None, :]
    pool = jnp.max(y5, axis=0)

    weff = weff_ref[...]
    beff = beff_ref[...]
    xfe = jnp.concatenate([pool[:, 0:3 * NF], othe_ref[...]], axis=1)
    xfo = jnp.concatenate([pool[:, 3 * NF:6 * NF], otho_ref[...]],
                          axis=1)
    le_ref[...] = jnp.dot(xfe, weff,
                          preferred_element_type=jnp.float32) + beff
    lo_ref[...] = jnp.dot(xfo, weff,
                          preferred_element_type=jnp.float32) + beff


def _tc_specs():
    in_specs = [
        pl.BlockSpec((HB, 128), lambda b: (b, 0)),
        pl.BlockSpec((HB, 128), lambda b: (b, 0)),
        pl.BlockSpec((NEG, HB, 128), lambda b: (0, b, 0)),
        pl.BlockSpec((L, HB, 128), lambda b: (0, b, 0)),
        pl.BlockSpec((HB, 9), lambda b: (b, 0)),
        pl.BlockSpec((HB, 9), lambda b: (b, 0)),
        pl.BlockSpec((5, 128, 6 * NF), lambda b: (0, 0, 0)),
        pl.BlockSpec((1, 6 * NF), lambda b: (0, 0)),
        pl.BlockSpec((L - FS[0] + 1, 6 * NF), lambda b: (0, 0)),
        pl.BlockSpec((69, 2), lambda b: (0, 0)),
        pl.BlockSpec((1, 2), lambda b: (0, 0)),
        pl.BlockSpec((128, 2), lambda b: (0, 0)),
    ]
    out_specs = [
        pl.BlockSpec((HB, 2), lambda b: (b, 0)),
        pl.BlockSpec((HB, 2), lambda b: (b, 0)),
        pl.BlockSpec((1, 1), lambda b: (0, 0)),
    ]
    out_shape = [
        jax.ShapeDtypeStruct((B // 2, 2), jnp.float32),
        jax.ShapeDtypeStruct((B // 2, 2), jnp.float32),
        jax.ShapeDtypeStruct((1, 1), jnp.float32),
    ]
    return (NB,), in_specs, out_specs, out_shape


def _dense(ein_pk, ectx_pk, eneg_pk, x_pk, othe, otho, w5, b120, msk,
           weff, beff, hmat):
    grid, in_specs, out_specs, out_shape = _tc_specs()
    return pl.pallas_call(
        _tc_body, grid=grid, in_specs=in_specs, out_specs=out_specs,
        out_shape=out_shape,
    )(ein_pk, ectx_pk, eneg_pk, x_pk, othe, otho, w5, b120, msk, weff,
      beff, hmat)


def _sc_gather_build():
    xpw = (L * B) // NW
    npw = (NEG * B) // NW
    bpw = B // NW
    nx = xpw // CH
    nn = npw // CH
    PCH = CH // 2

    mesh = plsc.VectorSubcoreMesh(core_axis_name="c", subcore_axis_name="s")

    @functools.partial(
        pl.kernel,
        mesh=mesh,
        compiler_params=pltpu.CompilerParams(use_tc_tiling_on_sc=False),
        out_type=[
            jax.ShapeDtypeStruct((B, D), jnp.float32),
            jax.ShapeDtypeStruct((L * B, D), jnp.float32),
            jax.ShapeDtypeStruct((B, D), jnp.float32),
            jax.ShapeDtypeStruct((NEG * B, D), jnp.float32),
        ],
        scratch_types=[
            pltpu.VMEM((xpw,), jnp.int32),
            pltpu.VMEM((npw,), jnp.int32),
            pltpu.VMEM((bpw,), jnp.int32),
            pltpu.VMEM((bpw,), jnp.int32),
            pltpu.VMEM((CH, D), jnp.float32),
            pltpu.VMEM((CH, D), jnp.float32),
            pltpu.VMEM((CH, D), jnp.float32),
            pltpu.VMEM((CH, D), jnp.float32),
            pltpu.SemaphoreType.DMA,
            pltpu.SemaphoreType.DMA,
            pltpu.SemaphoreType.DMA,
            pltpu.SemaphoreType.DMA,
        ],
    )
    def sc(emb_in_h, emb_ctx_h, iw_h, idsT_h, cw_h, negT_h,
           ein_o, x_o, ectx_o, eneg_o,
           ix_v, in_v, iw_v, ic_v, buf0, buf1, bufA, bufC,
           sem0, sem1, semA, semC):
        w = lax.axis_index("s") * 2 + lax.axis_index("c")

        pltpu.sync_copy(idsT_h.at[pl.ds(w * xpw, xpw)], ix_v)
        pltpu.sync_copy(negT_h.at[pl.ds(w * npw, npw)], in_v)
        pltpu.sync_copy(iw_h.at[pl.ds(w * bpw, bpw)], iw_v)
        pltpu.sync_copy(cw_h.at[pl.ds(w * bpw, bpw)], ic_v)

        cpA = pltpu.make_async_copy(emb_in_h.at[iw_v], bufA, semA)
        cpC = pltpu.make_async_copy(emb_ctx_h.at[ic_v], bufC, semC)
        cpA.start()
        cpC.start()

        bufs = (buf0, buf1)
        sems = (sem0, sem1)

        def run(table_h, idx_v, out_h, base, nch):
            def mk(slot, t):
                return pltpu.make_async_copy(
                    table_h.at[idx_v.at[pl.ds(t * CH, CH)]],
                    bufs[slot], sems[slot])

            mk(0, 0).start()
            mk(1, 1).start()

            def body(i, carry):
                t0 = i * 2
                for s in (0, 1):
                    t = t0 + s
                    mk(s, t).wait()
                    pltpu.sync_copy(bufs[s],
                                    out_h.at[pl.ds(base + t * CH, CH)])

                    @pl.when(t + 2 < nch)
                    def _():
                        mk(s, t + 2).start()
                return carry

            lax.fori_loop(0, nch // 2, body, 0)

        run(emb_in_h, ix_v, x_o, w * xpw, nx)

        cpA.wait()
        pltpu.sync_copy(bufA, ein_o.at[pl.ds(w * bpw, bpw)])
        cpC.wait()
        pltpu.sync_copy(bufC, ectx_o.at[pl.ds(w * bpw, bpw)])

        run(emb_ctx_h, in_v, eneg_o, w * npw, nn)

    return sc


def kernel(input_word, context_word, input_ids, other_features, emb_in,
           emb_ctx, neg_idx, conv_w0, conv_b0, conv_w1, conv_b1, conv_w2,
           conv_b2, fc_w0, fc_b0, fc_w1, fc_b1, fc_w2, fc_b2, fc_w3,
           fc_b3, fc_w4, fc_b4, fc_w5, fc_b5):
    iw = input_word.astype(jnp.int32)
    cw = context_word.astype(jnp.int32)
    idsT = input_ids.T.reshape(-1).astype(jnp.int32)
    negT = neg_idx.T.reshape(-1).astype(jnp.int32)

    sc = _sc_gather_build()
    ein_g, x_g, ectx_g, eneg_g = sc(emb_in, emb_ctx, iw, idsT, cw, negT)
    ein_pk = ein_g.reshape(B // 2, 128)
    ectx_pk = ectx_g.reshape(B // 2, 128)
    x3 = x_g.reshape(L, B // 2, 128)
    eneg3 = eneg_g.reshape(NEG, B // 2, 128)

    zc = jnp.zeros((D, NF), jnp.float32)
    zs = jnp.zeros((D, 3 * NF), jnp.float32)
    w5_list = []
    for k in range(FS[-1]):
        ek = jnp.concatenate(
            [conv_w0[:, :, k].T if k < FS[0] else zc,
             conv_w1[:, :, k].T if k < FS[1] else zc,
             conv_w2[:, :, k].T if k < FS[2] else zc], axis=1)
        w5_list.append(jnp.concatenate(
            [jnp.concatenate([ek, zs], axis=1),
             jnp.concatenate([zs, ek], axis=1)], axis=0))
    w5 = jnp.stack(w5_list, axis=0)

    b120 = jnp.concatenate([conv_b0, conv_b1, conv_b2,
                            conv_b0, conv_b1, conv_b2]).reshape(1, 6 * NF)
    LO = L - FS[0] + 1
    colt = jnp.tile(jnp.repeat(jnp.arange(3), NF), 2)
    louts = jnp.array([L - f + 1 for f in FS], jnp.int32)
    msk = (jnp.arange(LO)[:, None] < louts[colt][None, :]
           ).astype(jnp.float32)

    weff = fc_w0
    beff = fc_b0
    for wi, bi in ((fc_w1, fc_b1), (fc_w2, fc_b2), (fc_w3, fc_b3),
                   (fc_w4, fc_b4), (fc_w5, fc_b5)):
        weff = weff @ wi
        beff = beff @ wi + bi
    beff = beff.reshape(1, 2)

    hmat = (jnp.arange(128)[:, None] // 64
            == jnp.arange(2)[None, :]).astype(jnp.float32)

    othe = other_features[0::2]
    otho = other_features[1::2]

    le, lo, loss_arr = _dense(ein_pk, ectx_pk, eneg3, x3, othe, otho,
                              w5, b120, msk, weff, beff, hmat)
    logits = jnp.stack([le, lo], axis=1).reshape(B, 2)
    return (loss_arr[0, 0], logits)


# speedup vs baseline: 4.6603x; 1.0449x over previous
import functools

import jax
import jax.numpy as jnp
from jax import lax
from jax.experimental import pallas as pl
from jax.experimental.pallas import tpu as pltpu
from jax.experimental.pallas import tpu_sc as plsc

B = 4096
V = 100000
D = 64
L = 50
NEG = 10
NF = 20
FS = (3, 4, 5)
NCOL = sum(FS) * NF

BB = 128
HB = BB // 2
NB = B // BB
CH = 128
NW = 32


def _logsig(x):
    return jnp.minimum(x, 0.0) - jnp.log(1.0 + jnp.exp(-jnp.abs(x)))


def _tc_body(ein_ref, ectx_ref, eneg_ref, x_ref, othe_ref, otho_ref,
             w5_ref, b120_ref, msk_ref, weff_ref, beff_ref, h_ref,
             lo4_ref, loss_ref):
    b = pl.program_id(0)
    hmat = h_ref[...]

    ein = ein_ref[...]
    ectx = ectx_ref[...]
    prod = jnp.dot(ein * ectx, hmat,
                   preferred_element_type=jnp.float32)
    pos = _logsig(prod)

    eneg = eneg_ref[...]
    m = (eneg * ein[None, :, :]).reshape(NEG * HB, 128)
    nd = -jnp.dot(m, hmat, preferred_element_type=jnp.float32)
    noise = jnp.sum(_logsig(nd).reshape(NEG, HB, 2), axis=0)

    part = jnp.sum(pos + noise)

    @pl.when(b == 0)
    def _():
        loss_ref[...] = jnp.zeros((1, 1), jnp.float32)

    loss_ref[...] = loss_ref[...] + part

    @pl.when(b == NB - 1)
    def _():
        loss_ref[...] = loss_ref[...] * (-1.0 / B)

    x = x_ref[...]
    x2 = x.reshape(L * HB, 128)
    LO = L - FS[0] + 1
    y5 = None
    for k in range(FS[-1]):
        yk = jnp.dot(x2, w5_ref[k],
                     preferred_element_type=jnp.float32)
        yk = yk.reshape(L, HB, 2 * 3 * NF)
        if k + LO <= L:
            sh = yk[k:k + LO]
        else:
            sh = jnp.concatenate(
                [yk[k:L],
                 jnp.zeros((k + LO - L, HB, 2 * 3 * NF), jnp.float32)],
                axis=0)
        y5 = sh if y5 is None else y5 + sh

    y5 = jnp.maximum(y5 + b120_ref[...][None, :, :], 0.0)
    y5 = y5 * msk_ref[...][:, None, :]
    pool = jnp.max(y5, axis=0)

    weff = weff_ref[...]
    beff = beff_ref[...]
    xfe = jnp.concatenate([pool[:, 0:3 * NF], othe_ref[...]], axis=1)
    xfo = jnp.concatenate([pool[:, 3 * NF:6 * NF], otho_ref[...]],
                          axis=1)
    lo4_ref[...] = jnp.concatenate(
        [jnp.dot(xfe, weff, preferred_element_type=jnp.float32) + beff,
         jnp.dot(xfo, weff, preferred_element_type=jnp.float32) + beff],
        axis=1)


def _tc_specs():
    in_specs = [
        pl.BlockSpec((HB, 128), lambda b: (b, 0)),
        pl.BlockSpec((HB, 128), lambda b: (b, 0)),
        pl.BlockSpec((NEG, HB, 128), lambda b: (0, b, 0)),
        pl.BlockSpec((L, HB, 128), lambda b: (0, b, 0)),
        pl.BlockSpec((HB, 9), lambda b: (b, 0)),
        pl.BlockSpec((HB, 9), lambda b: (b, 0)),
        pl.BlockSpec((5, 128, 6 * NF), lambda b: (0, 0, 0)),
        pl.BlockSpec((1, 6 * NF), lambda b: (0, 0)),
        pl.BlockSpec((L - FS[0] + 1, 6 * NF), lambda b: (0, 0)),
        pl.BlockSpec((69, 2), lambda b: (0, 0)),
        pl.BlockSpec((1, 2), lambda b: (0, 0)),
        pl.BlockSpec((128, 2), lambda b: (0, 0)),
    ]
    out_specs = [
        pl.BlockSpec((HB, 4), lambda b: (b, 0)),
        pl.BlockSpec((1, 1), lambda b: (0, 0)),
    ]
    out_shape = [
        jax.ShapeDtypeStruct((B // 2, 4), jnp.float32),
        jax.ShapeDtypeStruct((1, 1), jnp.float32),
    ]
    return (NB,), in_specs, out_specs, out_shape


def _dense(ein_pk, ectx_pk, eneg_pk, x_pk, othe, otho, w5, b120, msk,
           weff, beff, hmat):
    grid, in_specs, out_specs, out_shape = _tc_specs()
    return pl.pallas_call(
        _tc_body, grid=grid, in_specs=in_specs, out_specs=out_specs,
        out_shape=out_shape,
    )(ein_pk, ectx_pk, eneg_pk, x_pk, othe, otho, w5, b120, msk, weff,
      beff, hmat)


def _sc_gather_build():
    bpw = B // NW
    mesh = plsc.VectorSubcoreMesh(core_axis_name="c", subcore_axis_name="s")

    @functools.partial(
        pl.kernel,
        mesh=mesh,
        compiler_params=pltpu.CompilerParams(use_tc_tiling_on_sc=False,
                                             needs_layout_passes=False),
        out_type=[
            jax.ShapeDtypeStruct((B, D), jnp.float32),
            jax.ShapeDtypeStruct((L * B, D), jnp.float32),
            jax.ShapeDtypeStruct((B, D), jnp.float32),
            jax.ShapeDtypeStruct((NEG * B, D), jnp.float32),
        ],
        scratch_types=[
            pltpu.VMEM((bpw * L,), jnp.int32),
            pltpu.VMEM((bpw * NEG,), jnp.int32),
            pltpu.VMEM((bpw,), jnp.int32),
            pltpu.VMEM((bpw,), jnp.int32),
            pltpu.VMEM((L * bpw,), jnp.int32),
            pltpu.VMEM((NEG * bpw,), jnp.int32),
            pltpu.VMEM((CH, D), jnp.float32),
            pltpu.VMEM((CH, D), jnp.float32),
            pltpu.VMEM((CH, D), jnp.float32),
            pltpu.VMEM((CH, D), jnp.float32),
            pltpu.SemaphoreType.DMA,
            pltpu.SemaphoreType.DMA,
            pltpu.SemaphoreType.DMA,
            pltpu.SemaphoreType.DMA,
        ],
    )
    def sc(emb_in_h, emb_ctx_h, iw_h, ids_h, cw_h, neg_h,
           ein_o, x_o, ectx_o, eneg_o,
           ids_v, neg_v, iw_v, ic_v, ix_v, in_v, buf0, buf1, bufA, bufC,
           sem0, sem1, semA, semC):
        w = lax.axis_index("s") * 2 + lax.axis_index("c")
        b0 = w * bpw

        pltpu.sync_copy(iw_h.at[pl.ds(b0, bpw)], iw_v)
        pltpu.sync_copy(cw_h.at[pl.ds(b0, bpw)], ic_v)
        pltpu.sync_copy(ids_h.at[pl.ds(b0 * L, bpw * L)], ids_v)
        pltpu.sync_copy(neg_h.at[pl.ds(b0 * NEG, bpw * NEG)], neg_v)

        cpA = pltpu.make_async_copy(emb_in_h.at[iw_v], bufA, semA)
        cpC = pltpu.make_async_copy(emb_ctx_h.at[ic_v], bufC, semC)
        cpA.start()
        cpC.start()

        lane = lax.broadcasted_iota(jnp.int32, (16,), 0)

        def transpose_slab(slab, dst, ncol):
            def col(p, carry):
                for j in range(bpw // 16):
                    idx = (lane + j * 16) * ncol + p
                    vals = plsc.load_gather(slab, [idx])
                    dst[pl.ds(p * bpw + j * 16, 16)] = vals
                return carry
            lax.fori_loop(0, ncol, col, 0)

        transpose_slab(ids_v, ix_v, L)
        transpose_slab(neg_v, in_v, NEG)

        bufs = (buf0, buf1)
        sems = (sem0, sem1)

        def run(table_h, idx_v, out_h, nch):
            def mk(slot, t):
                return pltpu.make_async_copy(
                    table_h.at[idx_v.at[pl.ds(t * CH, CH)]],
                    bufs[slot], sems[slot])

            mk(0, 0).start()
            mk(1, 1).start()

            def body(i, carry):
                t0 = i * 2
                for s in (0, 1):
                    t = t0 + s
                    mk(s, t).wait()
                    pltpu.sync_copy(bufs[s],
                                    out_h.at[pl.ds(t * B + b0, CH)])

                    @pl.when(t + 2 < nch)
                    def _():
                        mk(s, t + 2).start()
                return carry

            lax.fori_loop(0, nch // 2, body, 0)

        run(emb_in_h, ix_v, x_o, L)

        cpA.wait()
        pltpu.sync_copy(bufA, ein_o.at[pl.ds(b0, bpw)])
        cpC.wait()
        pltpu.sync_copy(bufC, ectx_o.at[pl.ds(b0, bpw)])

        run(emb_ctx_h, in_v, eneg_o, NEG)

    return sc


def kernel(input_word, context_word, input_ids, other_features, emb_in,
           emb_ctx, neg_idx, conv_w0, conv_b0, conv_w1, conv_b1, conv_w2,
           conv_b2, fc_w0, fc_b0, fc_w1, fc_b1, fc_w2, fc_b2, fc_w3,
           fc_b3, fc_w4, fc_b4, fc_w5, fc_b5):
    iw = input_word.astype(jnp.int32)
    cw = context_word.astype(jnp.int32)
    ids = input_ids.reshape(-1).astype(jnp.int32)
    neg = neg_idx.reshape(-1).astype(jnp.int32)

    sc = _sc_gather_build()
    ein_g, x_g, ectx_g, eneg_g = sc(emb_in, emb_ctx, iw, ids, cw, neg)
    ein_pk = ein_g.reshape(B // 2, 128)
    ectx_pk = ectx_g.reshape(B // 2, 128)
    x3 = x_g.reshape(L, B // 2, 128)
    eneg3 = eneg_g.reshape(NEG, B // 2, 128)

    zc = jnp.zeros((D, NF), jnp.float32)
    zs = jnp.zeros((D, 3 * NF), jnp.float32)
    w5_list = []
    for k in range(FS[-1]):
        ek = jnp.concatenate(
            [conv_w0[:, :, k].T if k < FS[0] else zc,
             conv_w1[:, :, k].T if k < FS[1] else zc,
             conv_w2[:, :, k].T if k < FS[2] else zc], axis=1)
        w5_list.append(jnp.concatenate(
            [jnp.concatenate([ek, zs], axis=1),
             jnp.concatenate([zs, ek], axis=1)], axis=0))
    w5 = jnp.stack(w5_list, axis=0)

    b120 = jnp.concatenate([conv_b0, conv_b1, conv_b2,
                            conv_b0, conv_b1, conv_b2]).reshape(1, 6 * NF)
    LO = L - FS[0] + 1
    colt = jnp.tile(jnp.repeat(jnp.arange(3), NF), 2)
    louts = jnp.array([L - f + 1 for f in FS], jnp.int32)
    msk = (jnp.arange(LO)[:, None] < louts[colt][None, :]
           ).astype(jnp.float32)

    weff = fc_w0
    beff = fc_b0
    for wi, bi in ((fc_w1, fc_b1), (fc_w2, fc_b2), (fc_w3, fc_b3),
                   (fc_w4, fc_b4), (fc_w5, fc_b5)):
        weff = weff @ wi
        beff = beff @ wi + bi
    beff = beff.reshape(1, 2)

    hmat = (jnp.arange(128)[:, None] // 64
            == jnp.arange(2)[None, :]).astype(jnp.float32)

    othe = other_features[0::2]
    otho = other_features[1::2]

    lo4, loss_arr = _dense(ein_pk, ectx_pk, eneg3, x3, othe, otho,
                           w5, b120, msk, weff, beff, hmat)
    logits = lo4.reshape(B, 2)
    return (loss_arr[0, 0], logits)


# speedup vs baseline: 4.6795x; 1.0041x over previous
import functools

import jax
import jax.numpy as jnp
from jax import lax
from jax.experimental import pallas as pl
from jax.experimental.pallas import tpu as pltpu
from jax.experimental.pallas import tpu_sc as plsc

B = 4096
V = 100000
D = 64
L = 50
NEG = 10
NF = 20
FS = (3, 4, 5)
NCOL = sum(FS) * NF

BB = 128
HB = BB // 2
NB = B // BB
CH = 128
NW = 32


def _logsig(x):
    return jnp.minimum(x, 0.0) - jnp.log(1.0 + jnp.exp(-jnp.abs(x)))


def _tc_body(ein_ref, ectx_ref, eneg_ref, x_ref, othe_ref, otho_ref,
             w5_ref, b120_ref, msk_ref, weff_ref, beff_ref, h_ref,
             lo4_ref, loss_ref):
    b = pl.program_id(0)
    hmat = h_ref[...]

    ein = ein_ref[...]
    ectx = ectx_ref[...]
    prod = jnp.dot(ein * ectx, hmat,
                   preferred_element_type=jnp.float32)
    pos = _logsig(prod)

    eneg = eneg_ref[...]
    m = (eneg * ein[None, :, :]).reshape(NEG * HB, 128)
    nd = -jnp.dot(m, hmat, preferred_element_type=jnp.float32)
    noise = jnp.sum(_logsig(nd).reshape(NEG, HB, 2), axis=0)

    part = jnp.sum(pos + noise)

    @pl.when(b == 0)
    def _():
        loss_ref[...] = jnp.zeros((1, 1), jnp.float32)

    loss_ref[...] = loss_ref[...] + part

    @pl.when(b == NB - 1)
    def _():
        loss_ref[...] = loss_ref[...] * (-1.0 / B)

    x = x_ref[...]
    x2 = x.reshape(L * HB, 128)
    LO = L - FS[0] + 1
    y5 = None
    for k in range(FS[-1]):
        yk = jnp.dot(x2, w5_ref[k],
                     preferred_element_type=jnp.float32)
        yk = yk.reshape(L, HB, 2 * 3 * NF)
        if k + LO <= L:
            sh = yk[k:k + LO]
        else:
            sh = jnp.concatenate(
                [yk[k:L],
                 jnp.zeros((k + LO - L, HB, 2 * 3 * NF), jnp.float32)],
                axis=0)
        y5 = sh if y5 is None else y5 + sh

    y5 = jnp.maximum(y5 + b120_ref[...][None, :, :], 0.0)
    y5 = y5 * msk_ref[...][:, None, :]
    pool = jnp.max(y5, axis=0)

    weff = weff_ref[...]
    beff = beff_ref[...]
    xfe = jnp.concatenate([pool[:, 0:3 * NF], othe_ref[...]], axis=1)
    xfo = jnp.concatenate([pool[:, 3 * NF:6 * NF], otho_ref[...]],
                          axis=1)
    lo4_ref[...] = jnp.concatenate(
        [jnp.dot(xfe, weff, preferred_element_type=jnp.float32) + beff,
         jnp.dot(xfo, weff, preferred_element_type=jnp.float32) + beff],
        axis=1)


def _tc_specs():
    in_specs = [
        pl.BlockSpec((HB, 128), lambda b: (b, 0)),
        pl.BlockSpec((HB, 128), lambda b: (b, 0)),
        pl.BlockSpec((NEG, HB, 128), lambda b: (0, b, 0)),
        pl.BlockSpec((L, HB, 128), lambda b: (0, b, 0)),
        pl.BlockSpec((HB, 9), lambda b: (b, 0)),
        pl.BlockSpec((HB, 9), lambda b: (b, 0)),
        pl.BlockSpec((5, 128, 6 * NF), lambda b: (0, 0, 0)),
        pl.BlockSpec((1, 6 * NF), lambda b: (0, 0)),
        pl.BlockSpec((L - FS[0] + 1, 6 * NF), lambda b: (0, 0)),
        pl.BlockSpec((69, 2), lambda b: (0, 0)),
        pl.BlockSpec((1, 2), lambda b: (0, 0)),
        pl.BlockSpec((128, 2), lambda b: (0, 0)),
    ]
    out_specs = [
        pl.BlockSpec((HB, 4), lambda b: (b, 0)),
        pl.BlockSpec((1, 1), lambda b: (0, 0)),
    ]
    out_shape = [
        jax.ShapeDtypeStruct((B // 2, 4), jnp.float32),
        jax.ShapeDtypeStruct((1, 1), jnp.float32),
    ]
    return (NB,), in_specs, out_specs, out_shape


def _dense(ein_pk, ectx_pk, eneg_pk, x_pk, othe, otho, w5, b120, msk,
           weff, beff, hmat):
    grid, in_specs, out_specs, out_shape = _tc_specs()
    return pl.pallas_call(
        _tc_body, grid=grid, in_specs=in_specs, out_specs=out_specs,
        out_shape=out_shape,
    )(ein_pk, ectx_pk, eneg_pk, x_pk, othe, otho, w5, b120, msk, weff,
      beff, hmat)


def _sc_gather_build(ncol):
    bpw = B // NW
    mesh = plsc.VectorSubcoreMesh(core_axis_name="c", subcore_axis_name="s")

    @functools.partial(
        pl.kernel,
        mesh=mesh,
        compiler_params=pltpu.CompilerParams(use_tc_tiling_on_sc=False,
                                             needs_layout_passes=False),
        out_type=[
            jax.ShapeDtypeStruct((B, D), jnp.float32),
            jax.ShapeDtypeStruct((ncol * B, D), jnp.float32),
        ],
        scratch_types=[
            pltpu.VMEM((bpw * ncol,), jnp.int32),
            pltpu.VMEM((bpw,), jnp.int32),
            pltpu.VMEM((ncol * bpw,), jnp.int32),
            pltpu.VMEM((CH, D), jnp.float32),
            pltpu.VMEM((CH, D), jnp.float32),
            pltpu.VMEM((CH, D), jnp.float32),
            pltpu.SemaphoreType.DMA,
            pltpu.SemaphoreType.DMA,
            pltpu.SemaphoreType.DMA,
        ],
    )
    def sc(emb_h, word_h, tab_h, word_o, rows_o,
           tab_v, iw_v, ix_v, buf0, buf1, bufA, sem0, sem1, semA):
        w = lax.axis_index("s") * 2 + lax.axis_index("c")
        b0 = w * bpw

        pltpu.sync_copy(word_h.at[pl.ds(b0, bpw)], iw_v)
        pltpu.sync_copy(tab_h.at[pl.ds(b0 * ncol, bpw * ncol)], tab_v)

        cpA = pltpu.make_async_copy(emb_h.at[iw_v], bufA, semA)
        cpA.start()

        lane = lax.broadcasted_iota(jnp.int32, (16,), 0)

        def col(p, carry):
            for j in range(bpw // 16):
                idx = (lane + j * 16) * ncol + p
                vals = plsc.load_gather(tab_v, [idx])
                ix_v[pl.ds(p * bpw + j * 16, 16)] = vals
            return carry
        lax.fori_loop(0, ncol, col, 0)

        bufs = (buf0, buf1)
        sems = (sem0, sem1)

        def mk(slot, t):
            return pltpu.make_async_copy(
                emb_h.at[ix_v.at[pl.ds(t * CH, CH)]],
                bufs[slot], sems[slot])

        mk(0, 0).start()
        mk(1, 1).start()

        def body(i, carry):
            t0 = i * 2
            for s in (0, 1):
                t = t0 + s
                mk(s, t).wait()
                pltpu.sync_copy(bufs[s], rows_o.at[pl.ds(t * B + b0, CH)])

                @pl.when(t + 2 < ncol)
                def _():
                    mk(s, t + 2).start()
            return carry

        lax.fori_loop(0, ncol // 2, body, 0)

        cpA.wait()
        pltpu.sync_copy(bufA, word_o.at[pl.ds(b0, bpw)])

    return sc


def kernel(input_word, context_word, input_ids, other_features, emb_in,
           emb_ctx, neg_idx, conv_w0, conv_b0, conv_w1, conv_b1, conv_w2,
           conv_b2, fc_w0, fc_b0, fc_w1, fc_b1, fc_w2, fc_b2, fc_w3,
           fc_b3, fc_w4, fc_b4, fc_w5, fc_b5):
    iw = input_word.astype(jnp.int32)
    cw = context_word.astype(jnp.int32)
    ids = input_ids.reshape(-1).astype(jnp.int32)
    neg = neg_idx.reshape(-1).astype(jnp.int32)

    ein_g, x_g = _sc_gather_build(L)(emb_in, iw, ids)
    ectx_g, eneg_g = _sc_gather_build(NEG)(emb_ctx, cw, neg)
    ein_pk = ein_g.reshape(B // 2, 128)
    ectx_pk = ectx_g.reshape(B // 2, 128)
    x3 = x_g.reshape(L, B // 2, 128)
    eneg3 = eneg_g.reshape(NEG, B // 2, 128)

    zc = jnp.zeros((D, NF), jnp.float32)
    zs = jnp.zeros((D, 3 * NF), jnp.float32)
    w5_list = []
    for k in range(FS[-1]):
        ek = jnp.concatenate(
            [conv_w0[:, :, k].T if k < FS[0] else zc,
             conv_w1[:, :, k].T if k < FS[1] else zc,
             conv_w2[:, :, k].T if k < FS[2] else zc], axis=1)
        w5_list.append(jnp.concatenate(
            [jnp.concatenate([ek, zs], axis=1),
             jnp.concatenate([zs, ek], axis=1)], axis=0))
    w5 = jnp.stack(w5_list, axis=0)

    b120 = jnp.concatenate([conv_b0, conv_b1, conv_b2,
                            conv_b0, conv_b1, conv_b2]).reshape(1, 6 * NF)
    LO = L - FS[0] + 1
    colt = jnp.tile(jnp.repeat(jnp.arange(3), NF), 2)
    louts = jnp.array([L - f + 1 for f in FS], jnp.int32)
    msk = (jnp.arange(LO)[:, None] < louts[colt][None, :]
           ).astype(jnp.float32)

    weff = fc_w0
    beff = fc_b0
    for wi, bi in ((fc_w1, fc_b1), (fc_w2, fc_b2), (fc_w3, fc_b3),
                   (fc_w4, fc_b4), (fc_w5, fc_b5)):
        weff = weff @ wi
        beff = beff @ wi + bi
    beff = beff.reshape(1, 2)

    hmat = (jnp.arange(128)[:, None] // 64
            == jnp.arange(2)[None, :]).astype(jnp.float32)

    othe = other_features[0::2]
    otho = other_features[1::2]

    lo4, loss_arr = _dense(ein_pk, ectx_pk, eneg3, x3, othe, otho,
                           w5, b120, msk, weff, beff, hmat)
    logits = lo4.reshape(B, 2)
    return (loss_arr[0, 0], logits)


# speedup vs baseline: 4.8116x; 1.0282x over previous
import functools

import jax
import jax.numpy as jnp
from jax import lax
from jax.experimental import pallas as pl
from jax.experimental.pallas import tpu as pltpu
from jax.experimental.pallas import tpu_sc as plsc

B = 4096
V = 100000
D = 64
L = 50
NEG = 10
NF = 20
FS = (3, 4, 5)
NCOL = sum(FS) * NF

BB = 128
HB = BB // 2
NB = B // BB
CH = 128
NW = 32


def _logsig(x):
    return jnp.minimum(x, 0.0) - jnp.log(1.0 + jnp.exp(-jnp.abs(x)))


def _tc_body(ein_ref, ectx_ref, eneg_ref, x_ref, othe_ref, otho_ref,
             w5_ref, b120_ref, msk_ref, weff_ref, beff_ref, h_ref,
             lo4_ref, loss_ref):
    b = pl.program_id(0)
    hmat = h_ref[...]

    ein = ein_ref[...]
    ectx = ectx_ref[...]
    prod = jnp.dot(ein * ectx, hmat,
                   preferred_element_type=jnp.float32)
    pos = _logsig(prod)

    eneg = eneg_ref[...]
    m = (eneg * ein[None, :, :]).reshape(NEG * HB, 128)
    nd = -jnp.dot(m, hmat, preferred_element_type=jnp.float32)
    noise = jnp.sum(_logsig(nd).reshape(NEG, HB, 2), axis=0)

    part = jnp.sum(pos + noise)

    @pl.when(b == 0)
    def _():
        loss_ref[...] = jnp.zeros((1, 1), jnp.float32)

    loss_ref[...] = loss_ref[...] + part

    @pl.when(b == NB - 1)
    def _():
        loss_ref[...] = loss_ref[...] * (-1.0 / B)

    x = x_ref[...]
    x2 = x.reshape(L * HB, 128)
    LO = L - FS[0] + 1
    y5 = None
    for k in range(FS[-1]):
        yk = jnp.dot(x2, w5_ref[k],
                     preferred_element_type=jnp.float32)
        yk = yk.reshape(L, HB, 2 * 3 * NF)
        if k + LO <= L:
            sh = yk[k:k + LO]
        else:
            sh = jnp.concatenate(
                [yk[k:L],
                 jnp.zeros((k + LO - L, HB, 2 * 3 * NF), jnp.float32)],
                axis=0)
        y5 = sh if y5 is None else y5 + sh

    y5 = jnp.maximum(y5 + b120_ref[...][None, :, :], 0.0)
    y5 = y5 * msk_ref[...][:, None, :]
    pool = jnp.max(y5, axis=0)

    weff = weff_ref[...]
    beff = beff_ref[...]
    xfe = jnp.concatenate([pool[:, 0:3 * NF], othe_ref[...]], axis=1)
    xfo = jnp.concatenate([pool[:, 3 * NF:6 * NF], otho_ref[...]],
                          axis=1)
    lo4_ref[...] = jnp.concatenate(
        [jnp.dot(xfe, weff, preferred_element_type=jnp.float32) + beff,
         jnp.dot(xfo, weff, preferred_element_type=jnp.float32) + beff],
        axis=1)


def _tc_specs():
    in_specs = [
        pl.BlockSpec((HB, 128), lambda b: (b, 0)),
        pl.BlockSpec((HB, 128), lambda b: (b, 0)),
        pl.BlockSpec((NEG, HB, 128), lambda b: (0, b, 0)),
        pl.BlockSpec((L, HB, 128), lambda b: (0, b, 0)),
        pl.BlockSpec((HB, 9), lambda b: (b, 0)),
        pl.BlockSpec((HB, 9), lambda b: (b, 0)),
        pl.BlockSpec((5, 128, 6 * NF), lambda b: (0, 0, 0)),
        pl.BlockSpec((1, 6 * NF), lambda b: (0, 0)),
        pl.BlockSpec((L - FS[0] + 1, 6 * NF), lambda b: (0, 0)),
        pl.BlockSpec((69, 2), lambda b: (0, 0)),
        pl.BlockSpec((1, 2), lambda b: (0, 0)),
        pl.BlockSpec((128, 2), lambda b: (0, 0)),
    ]
    out_specs = [
        pl.BlockSpec((HB, 4), lambda b: (b, 0)),
        pl.BlockSpec((1, 1), lambda b: (0, 0)),
    ]
    out_shape = [
        jax.ShapeDtypeStruct((B // 2, 4), jnp.float32),
        jax.ShapeDtypeStruct((1, 1), jnp.float32),
    ]
    return (NB,), in_specs, out_specs, out_shape


def _dense(ein_pk, ectx_pk, eneg_pk, x_pk, othe, otho, w5, b120, msk,
           weff, beff, hmat):
    grid, in_specs, out_specs, out_shape = _tc_specs()
    return pl.pallas_call(
        _tc_body, grid=grid, in_specs=in_specs, out_specs=out_specs,
        out_shape=out_shape,
    )(ein_pk, ectx_pk, eneg_pk, x_pk, othe, otho, w5, b120, msk, weff,
      beff, hmat)


def _sc_gather_build(ncol):
    bpw = B // NW
    mesh = plsc.VectorSubcoreMesh(core_axis_name="c", subcore_axis_name="s")

    G = 5
    NS = ncol // G

    @functools.partial(
        pl.kernel,
        mesh=mesh,
        compiler_params=pltpu.CompilerParams(use_tc_tiling_on_sc=False,
                                             needs_layout_passes=False),
        out_type=[
            jax.ShapeDtypeStruct((B, D), jnp.float32),
            jax.ShapeDtypeStruct((ncol, B, D), jnp.float32),
        ],
        scratch_types=[
            pltpu.VMEM((bpw * ncol,), jnp.int32),
            pltpu.VMEM((bpw,), jnp.int32),
            pltpu.VMEM((ncol * bpw,), jnp.int32),
            pltpu.VMEM((G, CH, D), jnp.float32),
            pltpu.VMEM((G, CH, D), jnp.float32),
            pltpu.VMEM((CH, D), jnp.float32),
            pltpu.SemaphoreType.DMA,
            pltpu.SemaphoreType.DMA,
            pltpu.SemaphoreType.DMA,
        ],
    )
    def sc(emb_h, word_h, tab_h, word_o, rows_o,
           tab_v, iw_v, ix_v, buf0, buf1, bufA, sem0, sem1, semA):
        w = lax.axis_index("s") * 2 + lax.axis_index("c")
        b0 = w * bpw

        pltpu.sync_copy(word_h.at[pl.ds(b0, bpw)], iw_v)
        pltpu.sync_copy(tab_h.at[pl.ds(b0 * ncol, bpw * ncol)], tab_v)

        cpA = pltpu.make_async_copy(emb_h.at[iw_v], bufA, semA)
        cpA.start()

        lane = lax.broadcasted_iota(jnp.int32, (16,), 0)

        def col(p, carry):
            for j in range(bpw // 16):
                idx = (lane + j * 16) * ncol + p
                vals = plsc.load_gather(tab_v, [idx])
                ix_v[pl.ds(p * bpw + j * 16, 16)] = vals
            return carry
        lax.fori_loop(0, ncol, col, 0)

        bufs = (buf0, buf1)
        sems = (sem0, sem1)

        def mk(slot, S, j):
            return pltpu.make_async_copy(
                emb_h.at[ix_v.at[pl.ds((S * G + j) * CH, CH)]],
                bufs[slot].at[j], sems[slot])

        def gstart(slot, S):
            for j in range(G):
                mk(slot, S, j).start()

        gstart(0, 0)
        gstart(1, 1)

        def body(i, carry):
            for s in (0, 1):
                S = i * 2 + s
                for j in range(G):
                    mk(s, S, j).wait()
                pltpu.sync_copy(
                    bufs[s],
                    rows_o.at[pl.ds(S * G, G), pl.ds(b0, CH)])

                @pl.when(S + 2 < NS)
                def _():
                    gstart(s, S + 2)
            return carry

        lax.fori_loop(0, NS // 2, body, 0)

        cpA.wait()
        pltpu.sync_copy(bufA, word_o.at[pl.ds(b0, bpw)])

    return sc


def kernel(input_word, context_word, input_ids, other_features, emb_in,
           emb_ctx, neg_idx, conv_w0, conv_b0, conv_w1, conv_b1, conv_w2,
           conv_b2, fc_w0, fc_b0, fc_w1, fc_b1, fc_w2, fc_b2, fc_w3,
           fc_b3, fc_w4, fc_b4, fc_w5, fc_b5):
    iw = input_word.astype(jnp.int32)
    cw = context_word.astype(jnp.int32)
    ids = input_ids.reshape(-1).astype(jnp.int32)
    neg = neg_idx.reshape(-1).astype(jnp.int32)

    ein_g, x_g = _sc_gather_build(L)(emb_in, iw, ids)
    ectx_g, eneg_g = _sc_gather_build(NEG)(emb_ctx, cw, neg)
    ein_pk = ein_g.reshape(B // 2, 128)
    ectx_pk = ectx_g.reshape(B // 2, 128)
    x3 = x_g.reshape(L, B // 2, 128)
    eneg3 = eneg_g.reshape(NEG, B // 2, 128)

    zc = jnp.zeros((D, NF), jnp.float32)
    zs = jnp.zeros((D, 3 * NF), jnp.float32)
    w5_list = []
    for k in range(FS[-1]):
        ek = jnp.concatenate(
            [conv_w0[:, :, k].T if k < FS[0] else zc,
             conv_w1[:, :, k].T if k < FS[1] else zc,
             conv_w2[:, :, k].T if k < FS[2] else zc], axis=1)
        w5_list.append(jnp.concatenate(
            [jnp.concatenate([ek, zs], axis=1),
             jnp.concatenate([zs, ek], axis=1)], axis=0))
    w5 = jnp.stack(w5_list, axis=0)

    b120 = jnp.concatenate([conv_b0, conv_b1, conv_b2,
                            conv_b0, conv_b1, conv_b2]).reshape(1, 6 * NF)
    LO = L - FS[0] + 1
    colt = jnp.tile(jnp.repeat(jnp.arange(3), NF), 2)
    louts = jnp.array([L - f + 1 for f in FS], jnp.int32)
    msk = (jnp.arange(LO)[:, None] < louts[colt][None, :]
           ).astype(jnp.float32)

    hp = jax.lax.Precision.HIGHEST
    weff = fc_w0
    beff = fc_b0
    for wi, bi in ((fc_w1, fc_b1), (fc_w2, fc_b2), (fc_w3, fc_b3),
                   (fc_w4, fc_b4), (fc_w5, fc_b5)):
        weff = jnp.dot(weff, wi, precision=hp)
        beff = jnp.dot(beff, wi, precision=hp) + bi
    beff = beff.reshape(1, 2)

    hmat = (jnp.arange(128)[:, None] // 64
            == jnp.arange(2)[None, :]).astype(jnp.float32)

    othe = other_features[0::2]
    otho = other_features[1::2]

    lo4, loss_arr = _dense(ein_pk, ectx_pk, eneg3, x3, othe, otho,
                           w5, b120, msk, weff, beff, hmat)
    logits = lo4.reshape(B, 2)
    return (loss_arr[0, 0], logits)
